# baseline (speedup 1.0000x reference)
import functools

import jax
import jax.numpy as jnp
from jax import lax
from jax.experimental import pallas as pl
from jax.experimental.pallas import tpu as pltpu
from jax.experimental.pallas import tpu_sc as plsc

N = 10000
E = 320000
D = 128

NC = 2
NS = 16
NW = NC * NS

E_PAD = 327680
N_PAD = 10240
ACC_PER_TILE = N_PAD // NS

L1 = 128
ROWS1 = E_PAD // L1
R1_C0 = 160
R1_C1 = 0
SEG1 = 8

L2 = 64
ROWS2 = E_PAD // L2
R2_C0 = 272
R2_C1 = 48
SEG2 = 16

_mesh = plsc.VectorSubcoreMesh(
    core_axis_name="c", subcore_axis_name="s", num_cores=NC, num_subcores=NS
)


@functools.partial(
    pl.kernel,
    out_type=jax.ShapeDtypeStruct((NC, N_PAD, D), jnp.float32),
    mesh=_mesh,
    scratch_types=[
        pltpu.VMEM((SEG1, L1), jnp.int32),
        pltpu.VMEM((SEG1, L1), jnp.int32),
        pltpu.VMEM((L1, D), jnp.float32),
        pltpu.VMEM((L1, D), jnp.float32),
        pltpu.VMEM_SHARED((N_PAD, D), jnp.float32),
        pltpu.SemaphoreType.DMA,
        pltpu.SemaphoreType.DMA,
        pltpu.SemaphoreType.DMA,
        pltpu.SemaphoreType.DMA,
    ],
)
def _sc_segsum(src_hbm, dst_hbm, x_hbm, z_hbm, out_hbm,
               sidx, didx, rows0, rows1, acc, semg0, semg1, sems0, sems1):
    c = lax.axis_index("c")
    s = lax.axis_index("s")
    base = s * ACC_PER_TILE
    row0 = jnp.where(c == 0, s * R1_C0, NS * R1_C0 + s * R1_C1)
    nseg = jnp.where(c == 0, R1_C0 // SEG1, R1_C1 // SEG1)

    pltpu.sync_copy(z_hbm.at[pl.ds(0, L1)], rows0)
    for t in range(ACC_PER_TILE // L1):
        pltpu.sync_copy(rows0, acc.at[pl.ds(base + t * L1, L1)])
    plsc.subcore_barrier()

    rows = (rows0, rows1)
    semg = (semg0, semg1)
    sems = (sems0, sems1)

    def gather(i, b):
        return pltpu.async_copy(x_hbm.at[sidx.at[i]], rows[b], semg[b])

    def scatter(i, b):
        return pltpu.async_copy(rows[b], acc.at[didx.at[i]], sems[b], add=True)

    def segment(h, carry):
        r0 = row0 + h * SEG1
        pltpu.sync_copy(src_hbm.at[pl.ds(r0, SEG1)], sidx)
        pltpu.sync_copy(dst_hbm.at[pl.ds(r0, SEG1)], didx)
        g0 = gather(0, 0)
        g1 = gather(1, 1)

        def pair(j, carry2):
            e = 2 * j
            g0.wait()
            sc0 = scatter(e, 0)
            g1.wait()
            sc1 = scatter(e + 1, 1)
            sc0.wait()
            gather(e + 2, 0)
            sc1.wait()
            gather(e + 3, 1)
            return carry2

        lax.fori_loop(0, SEG1 // 2 - 1, pair, 0)
        e = SEG1 - 2
        g0.wait()
        sc0 = scatter(e, 0)
        g1.wait()
        sc1 = scatter(e + 1, 1)
        sc0.wait()
        sc1.wait()
        return carry

    lax.fori_loop(0, nseg, segment, 0)
    plsc.subcore_barrier()
    pltpu.sync_copy(
        acc.at[pl.ds(base, ACC_PER_TILE)],
        out_hbm.at[c, pl.ds(base, ACC_PER_TILE)],
    )


@functools.partial(
    pl.kernel,
    out_type=jax.ShapeDtypeStruct((NC, N_PAD, D), jnp.float32),
    mesh=_mesh,
    scratch_types=[
        pltpu.VMEM((SEG2, L2), jnp.int32),
        pltpu.VMEM((SEG2, L2), jnp.int32),
        pltpu.VMEM((L2, D), jnp.float32),
        pltpu.VMEM((L2, D), jnp.float32),
        pltpu.VMEM((L2, D), jnp.float32),
        pltpu.VMEM((L2, D), jnp.float32),
        pltpu.VMEM_SHARED((N_PAD, D), jnp.float32),
        pltpu.SemaphoreType.DMA,
        pltpu.SemaphoreType.DMA,
        pltpu.SemaphoreType.DMA,
        pltpu.SemaphoreType.DMA,
    ],
)
def _sc_message(src_hbm, dst_hbm, ek_hbm, a_hbm, z_hbm, out_hbm,
                sidx, didx, k0b, k1b, a0b, a1b, acc,
                semg0, semg1, sems0, sems1):
    c = lax.axis_index("c")
    s = lax.axis_index("s")
    base = s * ACC_PER_TILE
    row0 = jnp.where(c == 0, s * R2_C0, NS * R2_C0 + s * R2_C1)
    nseg = jnp.where(c == 0, R2_C0 // SEG2, R2_C1 // SEG2)

    pltpu.sync_copy(z_hbm.at[pl.ds(0, L2)], k0b)
    for t in range(ACC_PER_TILE // L2):
        pltpu.sync_copy(k0b, acc.at[pl.ds(base + t * L2, L2)])
    plsc.subcore_barrier()

    kb = (k0b, k1b)
    ab = (a0b, a1b)
    semg = (semg0, semg1)
    sems = (sems0, sems1)

    def gather(i, b):
        gk = pltpu.async_copy(ek_hbm.at[sidx.at[i]], kb[b], semg[b])
        ga = pltpu.async_copy(a_hbm.at[didx.at[i]], ab[b], semg[b])
        return gk, ga

    def scatter(i, b):
        return pltpu.async_copy(ab[b], acc.at[didx.at[i]], sems[b], add=True)

    def compute(b):
        krows = kb[b]
        arows = ab[b]

        def relu_row(r, carry2):
            for q in range(D // 16):
                sl = pl.ds(q * 16, 16)
                arows[r, sl] = jnp.maximum(arows[r, sl] + krows[r, sl], 0.0)
            return carry2

        lax.fori_loop(0, L2, relu_row, 0)

    def segment(h, carry):
        r0 = row0 + h * SEG2
        pltpu.sync_copy(src_hbm.at[pl.ds(r0, SEG2)], sidx)
        pltpu.sync_copy(dst_hbm.at[pl.ds(r0, SEG2)], didx)
        gk0, ga0 = gather(0, 0)
        gk1, ga1 = gather(1, 1)

        def pair(j, carry2):
            e = 2 * j
            gk0.wait()
            ga0.wait()
            compute(0)
            sc0 = scatter(e, 0)
            gk1.wait()
            ga1.wait()
            compute(1)
            sc1 = scatter(e + 1, 1)
            sc0.wait()
            gather(e + 2, 0)
            sc1.wait()
            gather(e + 3, 1)
            return carry2

        lax.fori_loop(0, SEG2 // 2 - 1, pair, 0)
        e = SEG2 - 2
        gk0.wait()
        ga0.wait()
        compute(0)
        sc0 = scatter(e, 0)
        gk1.wait()
        ga1.wait()
        compute(1)
        sc1 = scatter(e + 1, 1)
        sc0.wait()
        sc1.wait()
        return carry

    lax.fori_loop(0, nseg, segment, 0)
    plsc.subcore_barrier()
    pltpu.sync_copy(
        acc.at[pl.ds(base, ACC_PER_TILE)],
        out_hbm.at[c, pl.ds(base, ACC_PER_TILE)],
    )


_BR = 2000


def _tc_qk_body(x_ref, wq_ref, wk_ref, bq_ref, eq_ref, ek_ref):
    xb = x_ref[...]
    eq_ref[...] = (
        jnp.dot(xb, wq_ref[...], preferred_element_type=jnp.float32,
                precision=lax.Precision.HIGHEST)
        + bq_ref[...]
    )
    ek_ref[...] = jnp.dot(xb, wk_ref[...], preferred_element_type=jnp.float32,
                          precision=lax.Precision.HIGHEST)


def _tc_neigh_body(p0_ref, p1_ref, eq_ref, wn_ref, a_ref):
    sb = p0_ref[0] + p1_ref[0]
    a_ref[...] = eq_ref[...] + jnp.dot(
        sb, wn_ref[...], preferred_element_type=jnp.float32,
        precision=lax.Precision.HIGHEST)


def _tc_out_body(f0_ref, f1_ref, wr_ref, br_ref, rst_ref):
    sb = f0_ref[0] + f1_ref[0]
    rst_ref[...] = (
        jnp.dot(sb, wr_ref[...], preferred_element_type=jnp.float32,
                precision=lax.Precision.HIGHEST)
        + br_ref[...]
    )


def _full(shape):
    return pl.BlockSpec(shape, lambda i: tuple(0 for _ in shape))


def kernel(x, edge_index, Wq, bq, Wk, Wn, Wr, br):
    src = edge_index[0]
    dst = edge_index[1]
    pad = E_PAD - E
    src_f = jnp.concatenate([src, jnp.zeros((pad,), jnp.int32)])
    pad_dst = N + (jnp.arange(pad, dtype=jnp.int32) % (N_PAD - N))
    dst_f = jnp.concatenate([dst, pad_dst])
    src_p1 = src_f.reshape(ROWS1, L1)
    dst_p1 = dst_f.reshape(ROWS1, L1)
    src_p2 = src_f.reshape(ROWS2, L2)
    dst_p2 = dst_f.reshape(ROWS2, L2)
    zeros = jnp.zeros((L1, D), jnp.float32)

    eq, ek = pl.pallas_call(
        _tc_qk_body,
        grid=(N // _BR,),
        in_specs=[
            pl.BlockSpec((_BR, D), lambda i: (i, 0)),
            _full((D, D)),
            _full((D, D)),
            _full((1, D)),
        ],
        out_specs=[
            pl.BlockSpec((_BR, D), lambda i: (i, 0)),
            pl.BlockSpec((_BR, D), lambda i: (i, 0)),
        ],
        out_shape=[
            jax.ShapeDtypeStruct((N, D), jnp.float32),
            jax.ShapeDtypeStruct((N, D), jnp.float32),
        ],
    )(x, Wq, Wk, bq.reshape(1, D))

    p = _sc_segsum(src_p1, dst_p1, x, zeros)

    a = pl.pallas_call(
        _tc_neigh_body,
        grid=(N // _BR,),
        in_specs=[
            pl.BlockSpec((1, _BR, D), lambda i: (0, i, 0)),
            pl.BlockSpec((1, _BR, D), lambda i: (1, i, 0)),
            pl.BlockSpec((_BR, D), lambda i: (i, 0)),
            _full((D, D)),
        ],
        out_specs=pl.BlockSpec((_BR, D), lambda i: (i, 0)),
        out_shape=jax.ShapeDtypeStruct((N, D), jnp.float32),
    )(p, p, eq, Wn)

    a_pad = jnp.concatenate([a, jnp.zeros((N_PAD - N, D), jnp.float32)])

    f = _sc_message(src_p2, dst_p2, ek, a_pad, zeros)

    rst = pl.pallas_call(
        _tc_out_body,
        grid=(N // _BR,),
        in_specs=[
            pl.BlockSpec((1, _BR, D), lambda i: (0, i, 0)),
            pl.BlockSpec((1, _BR, D), lambda i: (1, i, 0)),
            _full((D, D)),
            _full((1, D)),
        ],
        out_specs=pl.BlockSpec((_BR, D), lambda i: (i, 0)),
        out_shape=jax.ShapeDtypeStruct((N, D), jnp.float32),
    )(f, f, Wr, br.reshape(1, D))
    return rst


# speedup vs baseline: 2.4629x; 2.4629x over previous
import functools

import jax
import jax.numpy as jnp
from jax import lax
from jax.experimental import pallas as pl
from jax.experimental.pallas import tpu as pltpu
from jax.experimental.pallas import tpu_sc as plsc

N = 10000
E = 320000
D = 128

NC = 2
NS = 16
NW = NC * NS

E_PAD = 327680
N_PAD = 10240
ACC_PER_TILE = N_PAD // NS

L1 = 128
ROWS1 = E_PAD // L1
R1_C0 = 80
R1_C1 = 80
SEG1 = 8

L2 = 64
ROWS2 = E_PAD // L2
R2_C0 = 160
R2_C1 = 160
SEG2 = 16

_mesh = plsc.VectorSubcoreMesh(
    core_axis_name="c", subcore_axis_name="s", num_cores=NC, num_subcores=NS
)


@functools.partial(
    pl.kernel,
    out_type=jax.ShapeDtypeStruct((NC, N_PAD, D), jnp.float32),
    mesh=_mesh,
    scratch_types=[
        pltpu.VMEM((SEG1, L1), jnp.int32),
        pltpu.VMEM((SEG1, L1), jnp.int32),
        pltpu.VMEM((L1, D), jnp.float32),
        pltpu.VMEM((L1, D), jnp.float32),
        pltpu.VMEM_SHARED((N_PAD, D), jnp.float32),
        pltpu.SemaphoreType.DMA,
        pltpu.SemaphoreType.DMA,
        pltpu.SemaphoreType.DMA,
        pltpu.SemaphoreType.DMA,
    ],
)
def _sc_segsum(src_hbm, dst_hbm, x_hbm, z_hbm, out_hbm,
               sidx, didx, rows0, rows1, acc, semg0, semg1, sems0, sems1):
    c = lax.axis_index("c")
    s = lax.axis_index("s")
    base = s * ACC_PER_TILE
    row0 = jnp.where(c == 0, s * R1_C0, NS * R1_C0 + s * R1_C1)
    nseg = jnp.where(c == 0, R1_C0 // SEG1, R1_C1 // SEG1)

    pltpu.sync_copy(z_hbm.at[pl.ds(0, L1)], rows0)
    for t in range(ACC_PER_TILE // L1):
        pltpu.sync_copy(rows0, acc.at[pl.ds(base + t * L1, L1)])
    plsc.subcore_barrier()

    rows = (rows0, rows1)
    semg = (semg0, semg1)
    sems = (sems0, sems1)

    def gather(i, b):
        return pltpu.async_copy(x_hbm.at[sidx.at[i]], rows[b], semg[b])

    def scatter(i, b):
        return pltpu.async_copy(rows[b], acc.at[didx.at[i]], sems[b], add=True)

    def segment(h, carry):
        r0 = row0 + h * SEG1
        pltpu.sync_copy(src_hbm.at[pl.ds(r0, SEG1)], sidx)
        pltpu.sync_copy(dst_hbm.at[pl.ds(r0, SEG1)], didx)
        g0 = gather(0, 0)
        g1 = gather(1, 1)

        def pair(j, carry2):
            e = 2 * j
            g0.wait()
            sc0 = scatter(e, 0)
            g1.wait()
            sc1 = scatter(e + 1, 1)
            sc0.wait()
            gather(e + 2, 0)
            sc1.wait()
            gather(e + 3, 1)
            return carry2

        lax.fori_loop(0, SEG1 // 2 - 1, pair, 0)
        e = SEG1 - 2
        g0.wait()
        sc0 = scatter(e, 0)
        g1.wait()
        sc1 = scatter(e + 1, 1)
        sc0.wait()
        sc1.wait()
        return carry

    lax.fori_loop(0, nseg, segment, 0)
    plsc.subcore_barrier()
    pltpu.sync_copy(
        acc.at[pl.ds(base, ACC_PER_TILE)],
        out_hbm.at[c, pl.ds(base, ACC_PER_TILE)],
    )


@functools.partial(
    pl.kernel,
    out_type=jax.ShapeDtypeStruct((NC, N_PAD, D), jnp.float32),
    mesh=_mesh,
    scratch_types=[
        pltpu.VMEM((SEG2, L2), jnp.int32),
        pltpu.VMEM((SEG2, L2), jnp.int32),
        pltpu.VMEM((L2, D), jnp.float32),
        pltpu.VMEM((L2, D), jnp.float32),
        pltpu.VMEM((L2, D), jnp.float32),
        pltpu.VMEM((L2, D), jnp.float32),
        pltpu.VMEM_SHARED((N_PAD, D), jnp.float32),
        pltpu.SemaphoreType.DMA,
        pltpu.SemaphoreType.DMA,
        pltpu.SemaphoreType.DMA,
        pltpu.SemaphoreType.DMA,
    ],
)
def _sc_message(src_hbm, dst_hbm, ek_hbm, a_hbm, z_hbm, out_hbm,
                sidx, didx, k0b, k1b, a0b, a1b, acc,
                semg0, semg1, sems0, sems1):
    c = lax.axis_index("c")
    s = lax.axis_index("s")
    base = s * ACC_PER_TILE
    row0 = jnp.where(c == 0, s * R2_C0, NS * R2_C0 + s * R2_C1)
    nseg = jnp.where(c == 0, R2_C0 // SEG2, R2_C1 // SEG2)

    pltpu.sync_copy(z_hbm.at[pl.ds(0, L2)], k0b)
    for t in range(ACC_PER_TILE // L2):
        pltpu.sync_copy(k0b, acc.at[pl.ds(base + t * L2, L2)])
    plsc.subcore_barrier()

    kb = (k0b, k1b)
    ab = (a0b, a1b)
    semg = (semg0, semg1)
    sems = (sems0, sems1)

    def gather(i, b):
        gk = pltpu.async_copy(ek_hbm.at[sidx.at[i]], kb[b], semg[b])
        ga = pltpu.async_copy(a_hbm.at[didx.at[i]], ab[b], semg[b])
        return gk, ga

    def scatter(i, b):
        return pltpu.async_copy(ab[b], acc.at[didx.at[i]], sems[b], add=True)

    def compute(b):
        krows = kb[b]
        arows = ab[b]

        def relu_row(r, carry2):
            for q in range(D // 16):
                sl = pl.ds(q * 16, 16)
                arows[r, sl] = jnp.maximum(arows[r, sl] + krows[r, sl], 0.0)
            return carry2

        lax.fori_loop(0, L2, relu_row, 0)

    def segment(h, carry):
        r0 = row0 + h * SEG2
        pltpu.sync_copy(src_hbm.at[pl.ds(r0, SEG2)], sidx)
        pltpu.sync_copy(dst_hbm.at[pl.ds(r0, SEG2)], didx)
        gk0, ga0 = gather(0, 0)
        gk1, ga1 = gather(1, 1)

        def pair(j, carry2):
            e = 2 * j
            gk0.wait()
            ga0.wait()
            compute(0)
            sc0 = scatter(e, 0)
            gk1.wait()
            ga1.wait()
            compute(1)
            sc1 = scatter(e + 1, 1)
            sc0.wait()
            gather(e + 2, 0)
            sc1.wait()
            gather(e + 3, 1)
            return carry2

        lax.fori_loop(0, SEG2 // 2 - 1, pair, 0)
        e = SEG2 - 2
        gk0.wait()
        ga0.wait()
        compute(0)
        sc0 = scatter(e, 0)
        gk1.wait()
        ga1.wait()
        compute(1)
        sc1 = scatter(e + 1, 1)
        sc0.wait()
        sc1.wait()
        return carry

    lax.fori_loop(0, nseg, segment, 0)
    plsc.subcore_barrier()
    pltpu.sync_copy(
        acc.at[pl.ds(base, ACC_PER_TILE)],
        out_hbm.at[c, pl.ds(base, ACC_PER_TILE)],
    )


_BR = 2000


def _tc_qk_body(x_ref, wq_ref, wk_ref, bq_ref, eq_ref, ek_ref):
    xb = x_ref[...]
    eq_ref[...] = (
        jnp.dot(xb, wq_ref[...], preferred_element_type=jnp.float32,
                precision=lax.Precision.HIGHEST)
        + bq_ref[...]
    )
    ek_ref[...] = jnp.dot(xb, wk_ref[...], preferred_element_type=jnp.float32,
                          precision=lax.Precision.HIGHEST)


def _tc_neigh_body(p0_ref, p1_ref, eq_ref, wn_ref, a_ref):
    sb = p0_ref[0] + p1_ref[0]
    a_ref[...] = eq_ref[...] + jnp.dot(
        sb, wn_ref[...], preferred_element_type=jnp.float32,
        precision=lax.Precision.HIGHEST)


def _tc_out_body(f0_ref, f1_ref, wr_ref, br_ref, rst_ref):
    sb = f0_ref[0] + f1_ref[0]
    rst_ref[...] = (
        jnp.dot(sb, wr_ref[...], preferred_element_type=jnp.float32,
                precision=lax.Precision.HIGHEST)
        + br_ref[...]
    )


def _full(shape):
    return pl.BlockSpec(shape, lambda i: tuple(0 for _ in shape))


def kernel(x, edge_index, Wq, bq, Wk, Wn, Wr, br):
    src = edge_index[0]
    dst = edge_index[1]
    pad = E_PAD - E
    pad_src = jnp.arange(pad, dtype=jnp.int32) % N
    src_f = jnp.concatenate([src, pad_src])
    pad_dst = N + (jnp.arange(pad, dtype=jnp.int32) % (N_PAD - N))
    dst_f = jnp.concatenate([dst, pad_dst])
    src_p1 = src_f.reshape(ROWS1, L1)
    dst_p1 = dst_f.reshape(ROWS1, L1)
    src_p2 = src_f.reshape(ROWS2, L2)
    dst_p2 = dst_f.reshape(ROWS2, L2)
    zeros = jnp.zeros((L1, D), jnp.float32)

    eq, ek = pl.pallas_call(
        _tc_qk_body,
        grid=(N // _BR,),
        in_specs=[
            pl.BlockSpec((_BR, D), lambda i: (i, 0)),
            _full((D, D)),
            _full((D, D)),
            _full((1, D)),
        ],
        out_specs=[
            pl.BlockSpec((_BR, D), lambda i: (i, 0)),
            pl.BlockSpec((_BR, D), lambda i: (i, 0)),
        ],
        out_shape=[
            jax.ShapeDtypeStruct((N, D), jnp.float32),
            jax.ShapeDtypeStruct((N, D), jnp.float32),
        ],
    )(x, Wq, Wk, bq.reshape(1, D))

    p = _sc_segsum(src_p1, dst_p1, x, zeros)

    a = pl.pallas_call(
        _tc_neigh_body,
        grid=(N // _BR,),
        in_specs=[
            pl.BlockSpec((1, _BR, D), lambda i: (0, i, 0)),
            pl.BlockSpec((1, _BR, D), lambda i: (1, i, 0)),
            pl.BlockSpec((_BR, D), lambda i: (i, 0)),
            _full((D, D)),
        ],
        out_specs=pl.BlockSpec((_BR, D), lambda i: (i, 0)),
        out_shape=jax.ShapeDtypeStruct((N, D), jnp.float32),
    )(p, p, eq, Wn)

    a_pad = jnp.concatenate([a, jnp.zeros((N_PAD - N, D), jnp.float32)])

    f = _sc_message(src_p2, dst_p2, ek, a_pad, zeros)

    rst = pl.pallas_call(
        _tc_out_body,
        grid=(N // _BR,),
        in_specs=[
            pl.BlockSpec((1, _BR, D), lambda i: (0, i, 0)),
            pl.BlockSpec((1, _BR, D), lambda i: (1, i, 0)),
            _full((D, D)),
            _full((1, D)),
        ],
        out_specs=pl.BlockSpec((_BR, D), lambda i: (i, 0)),
        out_shape=jax.ShapeDtypeStruct((N, D), jnp.float32),
    )(f, f, Wr, br.reshape(1, D))
    return rst
